# Initial kernel scaffold; baseline (speedup 1.0000x reference)
import jax
import jax.numpy as jnp
from jax.experimental import pallas as pl


def kernel(x, core0, core1, core2):
    raise NotImplementedError("write your pallas kernel here")


# speedup vs baseline: 49.2288x; 49.2288x over previous
import jax
import jax.numpy as jnp
from jax import lax
from jax.experimental import pallas as pl
from jax.experimental.pallas import tpu as pltpu
from jax.experimental.pallas import tpu_sc as plsc

B = 16384 * 26
NC, NS = 2, 16
NW = NC * NS
BPW = B // NW
GROUP = 256
NGROUPS = BPW // GROUP
CHUNK = 128
NCHUNK = GROUP // CHUNK
VEC = 16


def _m12_body(c2a_ref, c2b_ref, c1_ref, outa_ref, outb_ref):
    c1 = c1_ref[...]
    outa_ref[...] = jnp.dot(c2a_ref[...], c1, preferred_element_type=jnp.float32)
    outb_ref[...] = jnp.dot(c2b_ref[...], c1, preferred_element_type=jnp.float32)


def _table_body(a_ref, b_ref, w4a_ref, w4b_ref, out_ref):
    out_ref[...] = (
        jnp.dot(a_ref[...], w4a_ref[0], preferred_element_type=jnp.float32)
        + jnp.dot(b_ref[...], w4b_ref[0], preferred_element_type=jnp.float32)
    )


def _build_table(c2a, c2b, c1flat, w4a, w4b):
    m12a, m12b = pl.pallas_call(
        _m12_body,
        out_shape=(
            jax.ShapeDtypeStruct((100, 6400), jnp.float32),
            jax.ShapeDtypeStruct((100, 6400), jnp.float32),
        ),
    )(c2a, c2b, c1flat)
    m12a = m12a.reshape(10000, 64)
    m12b = m12b.reshape(10000, 64)
    t4 = pl.pallas_call(
        _table_body,
        grid=(25,),
        in_specs=[
            pl.BlockSpec((10000, 64), lambda i: (0, 0)),
            pl.BlockSpec((10000, 64), lambda i: (0, 0)),
            pl.BlockSpec((1, 64, 128), lambda i: (i, 0, 0)),
            pl.BlockSpec((1, 64, 128), lambda i: (i, 0, 0)),
        ],
        out_specs=pl.BlockSpec((10000, 128), lambda i: (i, 0)),
        out_shape=jax.ShapeDtypeStruct((250000, 128), jnp.float32),
    )(m12a, m12b, w4a, w4b)
    return t4


def _sc_body(x_ref, t_ref, out_ref, idx_v, row_v, buf_a, buf_b, sem_a, sem_b):
    wid = lax.axis_index("s") * NC + lax.axis_index("c")
    base = wid * BPW
    pltpu.sync_copy(x_ref.at[pl.ds(base, BPW)], idx_v)

    def digit_body(j, carry):
        v = idx_v[pl.ds(j * VEC, VEC)]
        vf = v.astype(jnp.float32) + 0.5
        d0 = (vf * (1.0 / 10000.0)).astype(jnp.int32)
        rem = v - d0 * 10000
        d1 = ((rem.astype(jnp.float32) + 0.5) * (1.0 / 100.0)).astype(jnp.int32)
        d2 = rem - d1 * 100
        f12 = d2 * 100 + d1
        row_v[pl.ds(j * VEC, VEC)] = (
            (d0 >> 2) * 40000 + f12 * 4 + (d0 & 3)
        )
        return carry

    lax.fori_loop(0, BPW // VEC, digit_body, 0)

    def fire(g, buf, sem):
        for c in range(NCHUNK):
            pltpu.async_copy(
                t_ref.at[row_v.at[pl.ds(g * GROUP + c * CHUNK, CHUNK)]],
                buf.at[pl.ds(c * CHUNK, CHUNK)],
                sem,
            )

    def drain(buf, sem):
        for c in range(NCHUNK):
            pltpu.make_async_copy(
                t_ref.at[row_v.at[pl.ds(0, CHUNK)]],
                buf.at[pl.ds(c * CHUNK, CHUNK)],
                sem,
            ).wait()

    def put(g, buf):
        pltpu.sync_copy(buf, out_ref.at[pl.ds(base + g * GROUP, GROUP)])

    fire(0, buf_a, sem_a)
    fire(1, buf_b, sem_b)

    def group_body(it, carry):
        g = 2 * it
        drain(buf_a, sem_a)
        put(g, buf_a)
        fire(g + 2, buf_a, sem_a)
        drain(buf_b, sem_b)
        put(g + 1, buf_b)
        fire(g + 3, buf_b, sem_b)
        return carry

    lax.fori_loop(0, NGROUPS // 2 - 1, group_body, 0)
    drain(buf_a, sem_a)
    put(NGROUPS - 2, buf_a)
    drain(buf_b, sem_b)
    put(NGROUPS - 1, buf_b)


def _make_sc_lookup():
    return pl.kernel(
        _sc_body,
        out_type=jax.ShapeDtypeStruct((B, 32), jnp.float32),
        compiler_params=pltpu.CompilerParams(use_tc_tiling_on_sc=False),
        mesh=plsc.VectorSubcoreMesh(
            core_axis_name="c", subcore_axis_name="s",
            num_cores=NC, num_subcores=NS,
        ),
        scratch_types=[
            pltpu.VMEM((BPW,), jnp.int32),
            pltpu.VMEM((BPW,), jnp.int32),
            pltpu.VMEM((GROUP, 32), jnp.float32),
            pltpu.VMEM((GROUP, 32), jnp.float32),
            pltpu.SemaphoreType.DMA,
            pltpu.SemaphoreType.DMA,
        ],
    )


def kernel(x, core0, core1, core2):
    xf = x.reshape(-1)
    g0 = core0[0]
    c2 = core2[..., 0]
    c2a = c2[:, :, 0].T
    c2b = c2[:, :, 1].T
    c1flat = jnp.transpose(core1, (3, 1, 0, 2)).reshape(16, 6400)
    z = jnp.einsum("dar,xc->rxdac", g0, jnp.eye(4, dtype=jnp.float32))
    z4 = jnp.transpose(z.reshape(16, 4, 25, 4, 4, 4), (2, 0, 1, 3, 4, 5))
    zero = jnp.zeros_like(z4)
    w4a = jnp.stack([z4, zero], axis=-1).reshape(25, 64, 128)
    w4b = jnp.stack([zero, z4], axis=-1).reshape(25, 64, 128)

    t4 = _build_table(c2a, c2b, c1flat, w4a, w4b)
    t = t4.reshape(1000000, 32)
    out = _make_sc_lookup()(xf, t)
    return out.reshape(x.shape + (32,))


# speedup vs baseline: 49.6214x; 1.0080x over previous
import jax
import jax.numpy as jnp
from jax import lax
from jax.experimental import pallas as pl
from jax.experimental.pallas import tpu as pltpu
from jax.experimental.pallas import tpu_sc as plsc

B = 16384 * 26
NC, NS = 2, 16
NW = NC * NS
BPW = B // NW
GROUP = 256
NGROUPS = BPW // GROUP
CHUNK = 128
NCHUNK = GROUP // CHUNK
VEC = 16


def _m12_body(c2a_ref, c2b_ref, c1_ref, outa_ref, outb_ref):
    c1 = c1_ref[...]
    outa_ref[...] = jnp.dot(c2a_ref[...], c1, preferred_element_type=jnp.float32)
    outb_ref[...] = jnp.dot(c2b_ref[...], c1, preferred_element_type=jnp.float32)


def _table_body(m_ref, w_ref, out_ref):
    res = jnp.dot(m_ref[...], w_ref[0], preferred_element_type=jnp.float32)
    out_ref[0:10000, :] = res[:, 0:128]
    out_ref[10000:20000, :] = res[:, 128:256]


def _build_table(c2a, c2b, c1flat, w):
    m12a, m12b = pl.pallas_call(
        _m12_body,
        out_shape=(
            jax.ShapeDtypeStruct((100, 6400), jnp.float32),
            jax.ShapeDtypeStruct((100, 6400), jnp.float32),
        ),
    )(c2a, c2b, c1flat)
    m12cat = jnp.concatenate(
        [m12a.reshape(10000, 64), m12b.reshape(10000, 64)], axis=1
    )
    t4 = pl.pallas_call(
        _table_body,
        grid=(13,),
        in_specs=[
            pl.BlockSpec((10000, 128), lambda i: (0, 0)),
            pl.BlockSpec((1, 128, 256), lambda i: (i, 0, 0)),
        ],
        out_specs=pl.BlockSpec((20000, 128), lambda i: (i, 0)),
        out_shape=jax.ShapeDtypeStruct((260000, 128), jnp.float32),
    )(m12cat, w)
    return t4


def _sc_body(x_ref, t_ref, out_ref, idx_v, row_v, buf_a, buf_b, sem_a, sem_b):
    wid = lax.axis_index("s") * NC + lax.axis_index("c")
    base = wid * BPW
    pltpu.sync_copy(x_ref.at[pl.ds(base, BPW)], idx_v)

    def digit_body(j, carry):
        v = idx_v[pl.ds(j * VEC, VEC)]
        vf = v.astype(jnp.float32) + 0.5
        d0 = (vf * (1.0 / 10000.0)).astype(jnp.int32)
        rem = v - d0 * 10000
        d1 = ((rem.astype(jnp.float32) + 0.5) * (1.0 / 100.0)).astype(jnp.int32)
        d2 = rem - d1 * 100
        f12 = d2 * 100 + d1
        row_v[pl.ds(j * VEC, VEC)] = (
            (d0 >> 2) * 40000 + f12 * 4 + (d0 & 3)
        )
        return carry

    lax.fori_loop(0, BPW // VEC, digit_body, 0)

    def fire(g, buf, sem):
        for c in range(NCHUNK):
            pltpu.async_copy(
                t_ref.at[row_v.at[pl.ds(g * GROUP + c * CHUNK, CHUNK)]],
                buf.at[pl.ds(c * CHUNK, CHUNK)],
                sem,
            )

    def drain(buf, sem):
        for c in range(NCHUNK):
            pltpu.make_async_copy(
                t_ref.at[row_v.at[pl.ds(0, CHUNK)]],
                buf.at[pl.ds(c * CHUNK, CHUNK)],
                sem,
            ).wait()

    def put(g, buf):
        pltpu.sync_copy(buf, out_ref.at[pl.ds(base + g * GROUP, GROUP)])

    fire(0, buf_a, sem_a)
    fire(1, buf_b, sem_b)

    def group_body(it, carry):
        g = 2 * it
        drain(buf_a, sem_a)
        put(g, buf_a)
        fire(g + 2, buf_a, sem_a)
        drain(buf_b, sem_b)
        put(g + 1, buf_b)
        fire(g + 3, buf_b, sem_b)
        return carry

    lax.fori_loop(0, NGROUPS // 2 - 1, group_body, 0)
    drain(buf_a, sem_a)
    put(NGROUPS - 2, buf_a)
    drain(buf_b, sem_b)
    put(NGROUPS - 1, buf_b)


def _make_sc_lookup():
    return pl.kernel(
        _sc_body,
        out_type=jax.ShapeDtypeStruct((B, 32), jnp.float32),
        compiler_params=pltpu.CompilerParams(use_tc_tiling_on_sc=False),
        mesh=plsc.VectorSubcoreMesh(
            core_axis_name="c", subcore_axis_name="s",
            num_cores=NC, num_subcores=NS,
        ),
        scratch_types=[
            pltpu.VMEM((BPW,), jnp.int32),
            pltpu.VMEM((BPW,), jnp.int32),
            pltpu.VMEM((GROUP, 32), jnp.float32),
            pltpu.VMEM((GROUP, 32), jnp.float32),
            pltpu.SemaphoreType.DMA,
            pltpu.SemaphoreType.DMA,
        ],
    )


def kernel(x, core0, core1, core2):
    xf = x.reshape(-1)
    g0 = core0[0]
    c2 = core2[..., 0]
    c2a = c2[:, :, 0].T
    c2b = c2[:, :, 1].T
    c1flat = jnp.transpose(core1, (3, 1, 0, 2)).reshape(16, 6400)
    g0p = jnp.concatenate(
        [g0, jnp.zeros((4, 4, 16), jnp.float32)], axis=0
    ).reshape(13, 8, 4, 16)
    w = jnp.einsum(
        "pdar,xc,ye->pyrxdace",
        g0p,
        jnp.eye(4, dtype=jnp.float32),
        jnp.eye(2, dtype=jnp.float32),
    ).reshape(13, 128, 256)

    t4 = _build_table(c2a, c2b, c1flat, w)
    t = t4.reshape(1040000, 32)
    out = _make_sc_lookup()(xf, t)
    return out.reshape(x.shape + (32,))
